# baseline (speedup 1.0000x reference)
import functools

import jax
import jax.numpy as jnp
from jax import lax
from jax.experimental import pallas as pl
from jax.experimental.pallas import tpu as pltpu
from jax.experimental.pallas import tpu_sc as plsc

N = 10000
E = 320000
D = 128
HALF = 64
NC = 2
NS = 16
ROWS_PER_TILE = N // NS
EDGES_PER_TILE = E // NS
CHUNK = 80
NCHUNK = EDGES_PER_TILE // CHUNK
G = 5
NPH = 2
NSG = NCHUNK // (NPH * G)
ZROWS = 25
L = 16


def _sc_body(with_deg, *refs):
    if with_deg:
        (tab_hbm, src_hbm, dst_hbm, agg_hbm, deg_hbm,
         agg_s, deg_s, srcb, dstb, rowb, onesb, zbuf, zbuf16) = refs[:13]
        sems = refs[13:]
    else:
        (tab_hbm, src_hbm, dst_hbm, agg_hbm,
         agg_s, srcb, dstb, rowb, zbuf) = refs[:9]
        sems = refs[9:]
    dsem = sems[0]
    isems = sems[1:1 + NPH]
    gsems = sems[1 + NPH:1 + NPH + NPH * G]
    ssems = sems[1 + NPH + NPH * G:1 + NPH + 2 * NPH * G]
    zsem = sems[1 + NPH + 2 * NPH * G]
    if with_deg:
        zsem16 = sems[2 + NPH + 2 * NPH * G]

    c = lax.axis_index("c")
    s = lax.axis_index("s")
    row0 = s * ROWS_PER_TILE
    chunk0 = s * NCHUNK

    def _src_issue(p, g):
        pltpu.async_copy(src_hbm.at[pl.ds((chunk0 + g * G), G), :],
                         srcb.at[p], isems[p])

    def _src_wait_transform(p, g):
        pltpu.make_async_copy(src_hbm.at[pl.ds((chunk0 + g * G), G), :],
                              srcb.at[p], isems[p]).wait()
        for b in range(G):
            for j in range(CHUNK // L):
                v = srcb[p, b, pl.ds(j * L, L)]
                srcb[p, b, pl.ds(j * L, L)] = v + v + c

    pltpu.async_copy(dst_hbm.at[pl.ds(chunk0, NCHUNK), :], dstb, dsem)
    _src_issue(0, 0)

    def _zb(i, _):
        for j in range(HALF // L):
            zbuf[i, pl.ds(j * L, L)] = jnp.zeros((L,), jnp.float32)
        return 0
    lax.fori_loop(0, ZROWS, _zb, 0)

    def _zcopy(k, _):
        pltpu.async_copy(zbuf, agg_s.at[pl.ds(row0 + k * ZROWS, ZROWS), :],
                         zsem)
        return 0
    lax.fori_loop(0, ROWS_PER_TILE // ZROWS, _zcopy, 0)

    def _zdrain(k, _):
        pltpu.make_async_copy(zbuf,
                              agg_s.at[pl.ds(row0 + k * ZROWS, ZROWS), :],
                              zsem).wait()
        return 0

    if with_deg:
        if True:
            def _zb16(i, _):
                zbuf16[i] = jnp.zeros((L,), jnp.float32)
                return 0
            lax.fori_loop(0, ZROWS, _zb16, 0)

            def _zcopy16(k, _):
                pltpu.async_copy(zbuf16,
                                 deg_s.at[pl.ds(row0 + k * ZROWS, ZROWS), :],
                                 zsem16)
                return 0
            lax.fori_loop(0, ROWS_PER_TILE // ZROWS, _zcopy16, 0)

            def _zdrain16(k, _):
                pltpu.make_async_copy(
                    zbuf16, deg_s.at[pl.ds(row0 + k * ZROWS, ZROWS), :],
                    zsem16).wait()
                return 0
            lax.fori_loop(0, ROWS_PER_TILE // ZROWS, _zdrain16, 0)

            def _ob(i, _):
                onesb[i] = jnp.ones((L,), jnp.float32)
                return 0
            lax.fori_loop(0, CHUNK, _ob, 0)

    pltpu.make_async_copy(dst_hbm.at[pl.ds(chunk0, NCHUNK), :], dstb,
                          dsem).wait()
    lax.fori_loop(0, ROWS_PER_TILE // ZROWS, _zdrain, 0)

    plsc.subcore_barrier()

    def _scatter_issue(p, b, gidx):
        pltpu.async_copy(rowb.at[p, b], agg_s.at[dstb.at[gidx]],
                         ssems[p * G + b], add=True)
        if with_deg:
            @pl.when(c == p)
            def _():
                pltpu.async_copy(onesb, deg_s.at[dstb.at[gidx]],
                                 ssems[p * G + b], add=True)

    def _scatter_wait(p, b, gidx):
        pltpu.make_async_copy(rowb.at[p, b], agg_s.at[dstb.at[gidx]],
                              ssems[p * G + b]).wait()
        if with_deg:
            @pl.when(c == p)
            def _():
                pltpu.make_async_copy(onesb, deg_s.at[dstb.at[gidx]],
                                      ssems[p * G + b]).wait()

    def _gref(p, b):
        return tab_hbm.at[srcb.at[p, b]]

    def _outer(sg, _):
        for p in range(NPH):
            g = sg * NPH + p
            @pl.when(sg > 0)
            def _():
                for b in range(G):
                    _scatter_wait(p, b, (g - NPH) * G + b)
            @pl.when(g + 1 < NSG * NPH)
            def _():
                _src_issue(1 - p, g + 1)
            _src_wait_transform(p, g)
            for b in range(G):
                pltpu.async_copy(_gref(p, b), rowb.at[p, b],
                                 gsems[p * G + b])
            for b in range(G):
                pltpu.make_async_copy(_gref(p, b), rowb.at[p, b],
                                      gsems[p * G + b]).wait()
                _scatter_issue(p, b, g * G + b)
        return 0
    lax.fori_loop(0, NSG, _outer, 0)

    for p in range(NPH):
        g = (NSG - 1) * NPH + p
        for b in range(G):
            _scatter_wait(p, b, g * G + b)

    plsc.subcore_barrier()

    pltpu.sync_copy(agg_s.at[pl.ds(row0, ROWS_PER_TILE), :],
                    agg_hbm.at[c, pl.ds(row0, ROWS_PER_TILE), :])
    if with_deg:
        pltpu.sync_copy(deg_s.at[pl.ds(row0, ROWS_PER_TILE), :],
                        deg_hbm.at[c, pl.ds(row0, ROWS_PER_TILE), :])


def _make_sc_agg(with_deg):
    mesh = plsc.VectorSubcoreMesh(core_axis_name="c", subcore_axis_name="s")
    out_type = [jax.ShapeDtypeStruct((NC, N, HALF), jnp.float32)]
    scratch = [
        pltpu.VMEM_SHARED((N, HALF), jnp.float32),
    ]
    if with_deg:
        out_type.append(jax.ShapeDtypeStruct((NC, N, 16), jnp.float32))
        scratch.append(pltpu.VMEM_SHARED((N, 16), jnp.float32))
    scratch += [
        pltpu.VMEM((NPH, G, CHUNK), jnp.int32),
        pltpu.VMEM((NCHUNK, CHUNK), jnp.int32),
        pltpu.VMEM((NPH, G, CHUNK, HALF), jnp.float32),
    ]
    if with_deg:
        scratch.append(pltpu.VMEM((CHUNK, 16), jnp.float32))
    scratch.append(pltpu.VMEM((ZROWS, HALF), jnp.float32))
    if with_deg:
        scratch.append(pltpu.VMEM((ZROWS, 16), jnp.float32))
    for _ in range(2 + NPH + 2 * NPH * G + (1 if with_deg else 0)):
        scratch.append(pltpu.SemaphoreType.DMA)

    return pl.kernel(
        functools.partial(_sc_body, with_deg),
        out_type=tuple(out_type),
        mesh=mesh,
        scratch_types=tuple(scratch),
        compiler_params=pltpu.CompilerParams(use_tc_tiling_on_sc=False),
        name="sc_sage_agg_deg" if with_deg else "sc_sage_agg",
    )


_sc_agg_deg = _make_sc_agg(True)
_sc_agg = _make_sc_agg(False)


BN = 2000


def _tc_layer_body(apply_relu, aggp, xin, deg, wl, wr, b, out):
    agg = jnp.concatenate([aggp[0], aggp[1]], axis=-1)
    dsum = deg[0, :, 0:1] + deg[1, :, 0:1]
    dinv = 1.0 / jnp.maximum(dsum, 1.0)
    acc = jnp.dot(agg * dinv, wl[...], preferred_element_type=jnp.float32)
    acc = acc + jnp.dot(xin[...], wr[...], preferred_element_type=jnp.float32)
    acc = acc + b[...]
    if apply_relu:
        acc = jnp.maximum(acc, 0.0)
    out[...] = acc


def _make_tc_layer(apply_relu):
    return pl.pallas_call(
        functools.partial(_tc_layer_body, apply_relu),
        grid=(N // BN,),
        in_specs=[
            pl.BlockSpec((NC, BN, HALF), lambda i: (0, i, 0)),
            pl.BlockSpec((BN, D), lambda i: (i, 0)),
            pl.BlockSpec((NC, BN, 16), lambda i: (0, i, 0)),
            pl.BlockSpec((D, D), lambda i: (0, 0)),
            pl.BlockSpec((D, D), lambda i: (0, 0)),
            pl.BlockSpec((1, D), lambda i: (0, 0)),
        ],
        out_specs=pl.BlockSpec((BN, D), lambda i: (i, 0)),
        out_shape=jax.ShapeDtypeStruct((N, D), jnp.float32),
    )


_tc_layer1 = _make_tc_layer(True)
_tc_layer2 = _make_tc_layer(False)


def kernel(x, edge_index, W1l, W1r, b1, W2l, W2r, b2):
    src = edge_index[0].reshape(E // CHUNK, CHUNK)
    dst = edge_index[1].reshape(E // CHUNK, CHUNK)
    x2 = x.reshape(2 * N, HALF)
    agg1, deg16 = _sc_agg_deg(x2, src, dst)
    h = _tc_layer1(agg1, x, deg16, W1l, W1r, b1.reshape(1, D))
    (agg2,) = _sc_agg(h.reshape(2 * N, HALF), src, dst)
    out = _tc_layer2(agg2, h, deg16, W2l, W2r, b2.reshape(1, D))
    return out


# speedup vs baseline: 1.0211x; 1.0211x over previous
import functools

import jax
import jax.numpy as jnp
from jax import lax
from jax.experimental import pallas as pl
from jax.experimental.pallas import tpu as pltpu
from jax.experimental.pallas import tpu_sc as plsc

N = 10000
E = 320000
D = 128
HALF = 64
NC = 2
NS = 16
ROWS_PER_TILE = N // NS
EDGES_PER_TILE = E // NS
CHUNK = 80
NCHUNK = EDGES_PER_TILE // CHUNK
G = 5
NPH = 2
NSG = NCHUNK // (NPH * G)
ZROWS = 25
L = 16


def _sc_body(with_deg, *refs):
    if with_deg:
        (tab_hbm, src_hbm, dst_hbm, agg_hbm, deg_hbm,
         agg_s, deg_s, srcb, dstb, rowb, onesb, zbuf, zbuf16) = refs[:13]
        sems = refs[13:]
    else:
        (tab_hbm, src_hbm, dst_hbm, agg_hbm,
         agg_s, srcb, dstb, rowb, zbuf) = refs[:9]
        sems = refs[9:]
    dsem = sems[0]
    isems = sems[1:1 + NPH]
    gsems = sems[1 + NPH:1 + NPH + NPH * G]
    ssems = sems[1 + NPH + NPH * G:1 + NPH + 2 * NPH * G]
    zsem = sems[1 + NPH + 2 * NPH * G]
    if with_deg:
        zsem16 = sems[2 + NPH + 2 * NPH * G]

    c = lax.axis_index("c")
    s = lax.axis_index("s")
    row0 = s * ROWS_PER_TILE
    chunk0 = s * NCHUNK

    def _src_issue(p, g):
        pltpu.async_copy(src_hbm.at[pl.ds((chunk0 + g * G), G), :],
                         srcb.at[p], isems[p])

    def _src_wait_transform(p, g):
        pltpu.make_async_copy(src_hbm.at[pl.ds((chunk0 + g * G), G), :],
                              srcb.at[p], isems[p]).wait()
        for b in range(G):
            for j in range(CHUNK // L):
                v = srcb[p, b, pl.ds(j * L, L)]
                srcb[p, b, pl.ds(j * L, L)] = v + v + c

    pltpu.async_copy(dst_hbm.at[pl.ds(chunk0, NCHUNK), :], dstb, dsem)
    _src_issue(0, 0)

    def _zb(i, _):
        for j in range(HALF // L):
            zbuf[i, pl.ds(j * L, L)] = jnp.zeros((L,), jnp.float32)
        return 0
    lax.fori_loop(0, ZROWS, _zb, 0)

    def _zcopy(k, _):
        pltpu.async_copy(zbuf, agg_s.at[pl.ds(row0 + k * ZROWS, ZROWS), :],
                         zsem)
        return 0
    lax.fori_loop(0, ROWS_PER_TILE // ZROWS, _zcopy, 0)

    def _zdrain(k, _):
        pltpu.make_async_copy(zbuf,
                              agg_s.at[pl.ds(row0 + k * ZROWS, ZROWS), :],
                              zsem).wait()
        return 0

    if with_deg:
        @pl.when(c == 0)
        def _():
            def _zb16(i, _):
                zbuf16[i] = jnp.zeros((L,), jnp.float32)
                return 0
            lax.fori_loop(0, ZROWS, _zb16, 0)

            def _zcopy16(k, _):
                pltpu.async_copy(zbuf16,
                                 deg_s.at[pl.ds(row0 + k * ZROWS, ZROWS), :],
                                 zsem16)
                return 0
            lax.fori_loop(0, ROWS_PER_TILE // ZROWS, _zcopy16, 0)

            def _zdrain16(k, _):
                pltpu.make_async_copy(
                    zbuf16, deg_s.at[pl.ds(row0 + k * ZROWS, ZROWS), :],
                    zsem16).wait()
                return 0
            lax.fori_loop(0, ROWS_PER_TILE // ZROWS, _zdrain16, 0)

            def _ob(i, _):
                onesb[i] = jnp.ones((L,), jnp.float32)
                return 0
            lax.fori_loop(0, CHUNK, _ob, 0)

    pltpu.make_async_copy(dst_hbm.at[pl.ds(chunk0, NCHUNK), :], dstb,
                          dsem).wait()
    lax.fori_loop(0, ROWS_PER_TILE // ZROWS, _zdrain, 0)

    plsc.subcore_barrier()

    def _scatter_issue(p, b, gidx):
        pltpu.async_copy(rowb.at[p, b], agg_s.at[dstb.at[gidx]],
                         ssems[p * G + b], add=True)
        if with_deg:
            @pl.when(c == 0)
            def _():
                pltpu.async_copy(onesb, deg_s.at[dstb.at[gidx]],
                                 ssems[p * G + b], add=True)

    def _scatter_wait(p, b, gidx):
        pltpu.make_async_copy(rowb.at[p, b], agg_s.at[dstb.at[gidx]],
                              ssems[p * G + b]).wait()
        if with_deg:
            @pl.when(c == 0)
            def _():
                pltpu.make_async_copy(onesb, deg_s.at[dstb.at[gidx]],
                                      ssems[p * G + b]).wait()

    def _gref(p, b):
        return tab_hbm.at[srcb.at[p, b]]

    def _outer(sg, _):
        for p in range(NPH):
            g = sg * NPH + p
            @pl.when(sg > 0)
            def _():
                for b in range(G):
                    _scatter_wait(p, b, (g - NPH) * G + b)
            @pl.when(g + 1 < NSG * NPH)
            def _():
                _src_issue(1 - p, g + 1)
            _src_wait_transform(p, g)
            for b in range(G):
                pltpu.async_copy(_gref(p, b), rowb.at[p, b],
                                 gsems[p * G + b])
            for b in range(G):
                pltpu.make_async_copy(_gref(p, b), rowb.at[p, b],
                                      gsems[p * G + b]).wait()
                _scatter_issue(p, b, g * G + b)
        return 0
    lax.fori_loop(0, NSG, _outer, 0)

    for p in range(NPH):
        g = (NSG - 1) * NPH + p
        for b in range(G):
            _scatter_wait(p, b, g * G + b)

    plsc.subcore_barrier()

    pltpu.sync_copy(agg_s.at[pl.ds(row0, ROWS_PER_TILE), :],
                    agg_hbm.at[c, pl.ds(row0, ROWS_PER_TILE), :])
    if with_deg:
        @pl.when(c == 0)
        def _():
            pltpu.sync_copy(deg_s.at[pl.ds(row0, ROWS_PER_TILE), :],
                            deg_hbm.at[pl.ds(row0, ROWS_PER_TILE), :])


def _make_sc_agg(with_deg):
    mesh = plsc.VectorSubcoreMesh(core_axis_name="c", subcore_axis_name="s")
    out_type = [jax.ShapeDtypeStruct((NC, N, HALF), jnp.float32)]
    scratch = [
        pltpu.VMEM_SHARED((N, HALF), jnp.float32),
    ]
    if with_deg:
        out_type.append(jax.ShapeDtypeStruct((N, 16), jnp.float32))
        scratch.append(pltpu.VMEM_SHARED((N, 16), jnp.float32))
    scratch += [
        pltpu.VMEM((NPH, G, CHUNK), jnp.int32),
        pltpu.VMEM((NCHUNK, CHUNK), jnp.int32),
        pltpu.VMEM((NPH, G, CHUNK, HALF), jnp.float32),
    ]
    if with_deg:
        scratch.append(pltpu.VMEM((CHUNK, 16), jnp.float32))
    scratch.append(pltpu.VMEM((ZROWS, HALF), jnp.float32))
    if with_deg:
        scratch.append(pltpu.VMEM((ZROWS, 16), jnp.float32))
    for _ in range(2 + NPH + 2 * NPH * G + (1 if with_deg else 0)):
        scratch.append(pltpu.SemaphoreType.DMA)

    return pl.kernel(
        functools.partial(_sc_body, with_deg),
        out_type=tuple(out_type),
        mesh=mesh,
        scratch_types=tuple(scratch),
        compiler_params=pltpu.CompilerParams(use_tc_tiling_on_sc=False),
        name="sc_sage_agg_deg" if with_deg else "sc_sage_agg",
    )


_sc_agg_deg = _make_sc_agg(True)
_sc_agg = _make_sc_agg(False)


BN = 2000


def _tc_layer_body(apply_relu, aggp, xin, deg, wl, wr, b, out):
    agg = jnp.concatenate([aggp[0], aggp[1]], axis=-1)
    dinv = 1.0 / jnp.maximum(deg[:, 0:1], 1.0)
    acc = jnp.dot(agg * dinv, wl[...], preferred_element_type=jnp.float32)
    acc = acc + jnp.dot(xin[...], wr[...], preferred_element_type=jnp.float32)
    acc = acc + b[...]
    if apply_relu:
        acc = jnp.maximum(acc, 0.0)
    out[...] = acc


def _make_tc_layer(apply_relu):
    return pl.pallas_call(
        functools.partial(_tc_layer_body, apply_relu),
        grid=(N // BN,),
        in_specs=[
            pl.BlockSpec((NC, BN, HALF), lambda i: (0, i, 0)),
            pl.BlockSpec((BN, D), lambda i: (i, 0)),
            pl.BlockSpec((BN, 16), lambda i: (i, 0)),
            pl.BlockSpec((D, D), lambda i: (0, 0)),
            pl.BlockSpec((D, D), lambda i: (0, 0)),
            pl.BlockSpec((1, D), lambda i: (0, 0)),
        ],
        out_specs=pl.BlockSpec((BN, D), lambda i: (i, 0)),
        out_shape=jax.ShapeDtypeStruct((N, D), jnp.float32),
    )


_tc_layer1 = _make_tc_layer(True)
_tc_layer2 = _make_tc_layer(False)


def kernel(x, edge_index, W1l, W1r, b1, W2l, W2r, b2):
    src = edge_index[0].reshape(E // CHUNK, CHUNK)
    dst = edge_index[1].reshape(E // CHUNK, CHUNK)
    x2 = x.reshape(2 * N, HALF)
    agg1, deg16 = _sc_agg_deg(x2, src, dst)
    h = _tc_layer1(agg1, x, deg16, W1l, W1r, b1.reshape(1, D))
    (agg2,) = _sc_agg(h.reshape(2 * N, HALF), src, dst)
    out = _tc_layer2(agg2, h, deg16, W2l, W2r, b2.reshape(1, D))
    return out


# speedup vs baseline: 1.0283x; 1.0071x over previous
import functools

import jax
import jax.numpy as jnp
from jax import lax
from jax.experimental import pallas as pl
from jax.experimental.pallas import tpu as pltpu
from jax.experimental.pallas import tpu_sc as plsc

N = 10000
E = 320000
D = 128
HALF = 64
NC = 2
NS = 16
ROWS_PER_TILE = N // NS
EDGES_PER_TILE = E // NS
CHUNK = 80
NCHUNK = EDGES_PER_TILE // CHUNK
G = 5
NPH = 2
NSG = NCHUNK // (NPH * G)
ZROWS = 25
L = 16


def _sc_body(with_deg, *refs):
    if with_deg:
        (tab_hbm, src_hbm, dst_hbm, agg_hbm, deg_hbm,
         agg_s, deg_s, srcb, dstb, rowb, onesb, zbuf, zbuf16) = refs[:13]
        sems = refs[13:]
    else:
        (tab_hbm, src_hbm, dst_hbm, agg_hbm,
         agg_s, srcb, dstb, rowb, zbuf) = refs[:9]
        sems = refs[9:]
    dsem = sems[0]
    isems = sems[1:1 + NPH]
    gsems = sems[1 + NPH:1 + NPH + NPH * G]
    ssems = sems[1 + NPH + NPH * G:1 + NPH + 2 * NPH * G]
    zsem = sems[1 + NPH + 2 * NPH * G]
    if with_deg:
        zsem16 = sems[2 + NPH + 2 * NPH * G]

    c = lax.axis_index("c")
    s = lax.axis_index("s")
    row0 = s * ROWS_PER_TILE
    chunk0 = s * NCHUNK

    def _src_issue(p, g):
        pltpu.async_copy(src_hbm.at[pl.ds((chunk0 + g * G), G), :],
                         srcb.at[p], isems[p])

    def _src_wait_transform(p, g):
        pltpu.make_async_copy(src_hbm.at[pl.ds((chunk0 + g * G), G), :],
                              srcb.at[p], isems[p]).wait()
        for b in range(G):
            for j in range(CHUNK // L):
                v = srcb[p, b, pl.ds(j * L, L)]
                srcb[p, b, pl.ds(j * L, L)] = v + v + c

    pltpu.async_copy(dst_hbm.at[pl.ds(chunk0, NCHUNK), :], dstb, dsem)
    _src_issue(0, 0)

    def _zb(i, _):
        for j in range(HALF // L):
            zbuf[i, pl.ds(j * L, L)] = jnp.zeros((L,), jnp.float32)
        return 0
    lax.fori_loop(0, ZROWS, _zb, 0)

    def _zcopy(k, _):
        pltpu.async_copy(zbuf, agg_s.at[pl.ds(row0 + k * ZROWS, ZROWS), :],
                         zsem)
        return 0
    lax.fori_loop(0, ROWS_PER_TILE // ZROWS, _zcopy, 0)

    def _zdrain(k, _):
        pltpu.make_async_copy(zbuf,
                              agg_s.at[pl.ds(row0 + k * ZROWS, ZROWS), :],
                              zsem).wait()
        return 0

    if with_deg:
        @pl.when(c == 0)
        def _():
            def _zb16(i, _):
                zbuf16[i] = jnp.zeros((L,), jnp.float32)
                return 0
            lax.fori_loop(0, ZROWS, _zb16, 0)

            def _zcopy16(k, _):
                pltpu.async_copy(zbuf16,
                                 deg_s.at[pl.ds(row0 + k * ZROWS, ZROWS), :],
                                 zsem16)
                return 0
            lax.fori_loop(0, ROWS_PER_TILE // ZROWS, _zcopy16, 0)

            def _zdrain16(k, _):
                pltpu.make_async_copy(
                    zbuf16, deg_s.at[pl.ds(row0 + k * ZROWS, ZROWS), :],
                    zsem16).wait()
                return 0
            lax.fori_loop(0, ROWS_PER_TILE // ZROWS, _zdrain16, 0)

            def _ob(i, _):
                onesb[i] = jnp.ones((L,), jnp.float32)
                return 0
            lax.fori_loop(0, CHUNK, _ob, 0)

    pltpu.make_async_copy(dst_hbm.at[pl.ds(chunk0, NCHUNK), :], dstb,
                          dsem).wait()
    lax.fori_loop(0, ROWS_PER_TILE // ZROWS, _zdrain, 0)

    plsc.subcore_barrier()

    def _scatter_issue(p, b, gidx):
        pltpu.async_copy(rowb.at[p, b], agg_s.at[dstb.at[gidx]],
                         ssems[p * G + b], add=True)
        if with_deg:
            @pl.when(c == 0)
            def _():
                pltpu.async_copy(onesb, deg_s.at[dstb.at[gidx]],
                                 ssems[p * G + b], add=True)

    def _scatter_wait(p, b, gidx):
        pltpu.make_async_copy(rowb.at[p, b], agg_s.at[dstb.at[gidx]],
                              ssems[p * G + b]).wait()
        if with_deg:
            @pl.when(c == 0)
            def _():
                pltpu.make_async_copy(onesb, deg_s.at[dstb.at[gidx]],
                                      ssems[p * G + b]).wait()

    def _gref(p, b):
        return tab_hbm.at[srcb.at[p, b]]

    def _outer(sg, _):
        for p in range(NPH):
            g = sg * NPH + p
            @pl.when(sg > 0)
            def _():
                for b in range(G):
                    _scatter_wait(p, b, (g - NPH) * G + b)
            @pl.when(g + 1 < NSG * NPH)
            def _():
                _src_issue(1 - p, g + 1)
            _src_wait_transform(p, g)
            for b in range(G):
                pltpu.async_copy(_gref(p, b), rowb.at[p, b],
                                 gsems[p * G + b])
            for b in range(G):
                pltpu.make_async_copy(_gref(p, b), rowb.at[p, b],
                                      gsems[p * G + b]).wait()
                _scatter_issue(p, b, g * G + b)
        return 0
    lax.fori_loop(0, NSG, _outer, 0)

    for p in range(NPH):
        g = (NSG - 1) * NPH + p
        for b in range(G):
            _scatter_wait(p, b, g * G + b)

    plsc.subcore_barrier()

    pltpu.sync_copy(agg_s.at[pl.ds(row0, ROWS_PER_TILE), :],
                    agg_hbm.at[c, pl.ds(row0, ROWS_PER_TILE), :])
    if with_deg:
        @pl.when(c == 0)
        def _():
            pltpu.sync_copy(deg_s.at[pl.ds(row0, ROWS_PER_TILE), :],
                            deg_hbm.at[pl.ds(row0, ROWS_PER_TILE), :])


def _make_sc_agg(with_deg):
    mesh = plsc.VectorSubcoreMesh(core_axis_name="c", subcore_axis_name="s")
    out_type = [jax.ShapeDtypeStruct((NC, N, HALF), jnp.float32)]
    scratch = [
        pltpu.VMEM_SHARED((N, HALF), jnp.float32),
    ]
    if with_deg:
        out_type.append(jax.ShapeDtypeStruct((N, 16), jnp.float32))
        scratch.append(pltpu.VMEM_SHARED((N, 16), jnp.float32))
    scratch += [
        pltpu.VMEM((NPH, G, CHUNK), jnp.int32),
        pltpu.VMEM((NCHUNK, CHUNK), jnp.int32),
        pltpu.VMEM((NPH, G, CHUNK, HALF), jnp.float32),
    ]
    if with_deg:
        scratch.append(pltpu.VMEM((CHUNK, 16), jnp.float32))
    scratch.append(pltpu.VMEM((ZROWS, HALF), jnp.float32))
    if with_deg:
        scratch.append(pltpu.VMEM((ZROWS, 16), jnp.float32))
    for _ in range(2 + NPH + 2 * NPH * G + (1 if with_deg else 0)):
        scratch.append(pltpu.SemaphoreType.DMA)

    return pl.kernel(
        functools.partial(_sc_body, with_deg),
        out_type=tuple(out_type),
        mesh=mesh,
        scratch_types=tuple(scratch),
        compiler_params=pltpu.CompilerParams(use_tc_tiling_on_sc=False),
        name="sc_sage_agg_deg" if with_deg else "sc_sage_agg",
    )


_sc_agg_deg = _make_sc_agg(True)
_sc_agg = _make_sc_agg(False)


BN = 5000


def _tc_layer_body(apply_relu, aggp, xin, deg, wl, wr, b, out):
    agg = jnp.concatenate([aggp[0], aggp[1]], axis=-1)
    dinv = 1.0 / jnp.maximum(deg[:, 0:1], 1.0)
    acc = jnp.dot(agg * dinv, wl[...], preferred_element_type=jnp.float32)
    acc = acc + jnp.dot(xin[...], wr[...], preferred_element_type=jnp.float32)
    acc = acc + b[...]
    if apply_relu:
        acc = jnp.maximum(acc, 0.0)
    out[...] = acc


def _make_tc_layer(apply_relu):
    return pl.pallas_call(
        functools.partial(_tc_layer_body, apply_relu),
        grid=(N // BN,),
        in_specs=[
            pl.BlockSpec((NC, BN, HALF), lambda i: (0, i, 0)),
            pl.BlockSpec((BN, D), lambda i: (i, 0)),
            pl.BlockSpec((BN, 16), lambda i: (i, 0)),
            pl.BlockSpec((D, D), lambda i: (0, 0)),
            pl.BlockSpec((D, D), lambda i: (0, 0)),
            pl.BlockSpec((1, D), lambda i: (0, 0)),
        ],
        out_specs=pl.BlockSpec((BN, D), lambda i: (i, 0)),
        out_shape=jax.ShapeDtypeStruct((N, D), jnp.float32),
    )


_tc_layer1 = _make_tc_layer(True)
_tc_layer2 = _make_tc_layer(False)


def kernel(x, edge_index, W1l, W1r, b1, W2l, W2r, b2):
    src = edge_index[0].reshape(E // CHUNK, CHUNK)
    dst = edge_index[1].reshape(E // CHUNK, CHUNK)
    x2 = x.reshape(2 * N, HALF)
    agg1, deg16 = _sc_agg_deg(x2, src, dst)
    h = _tc_layer1(agg1, x, deg16, W1l, W1r, b1.reshape(1, D))
    (agg2,) = _sc_agg(h.reshape(2 * N, HALF), src, dst)
    out = _tc_layer2(agg2, h, deg16, W2l, W2r, b2.reshape(1, D))
    return out
